# baseline (speedup 1.0000x reference)
import jax
import jax.numpy as jnp
from jax.experimental import pallas as pl

H = 512
W = 512
NPH = 32
NPW = 32
PH = H // NPH
PW = W // NPW
NP = NPH * NPW
C = 192

ROWF = PW * C
WCOLS = NPW * ROWF


def _copy_body(x_ref, o_ref):
    o_ref[...] = x_ref[...]


def kernel(x, mesh_pos, batch_idx):
    x2 = x.reshape(H, WCOLS)
    out = pl.pallas_call(
        _copy_body,
        grid=(NPH, NPW),
        in_specs=[pl.BlockSpec((PH, ROWF), lambda i, j: (i, j))],
        out_specs=pl.BlockSpec((PH, ROWF), lambda i, j: (i * NPW + j, 0)),
        out_shape=jax.ShapeDtypeStruct((NP * PH, ROWF), jnp.float32),
    )(x2)
    return out.reshape(1, NP, PH, PW, C)


# speedup vs baseline: 1.5075x; 1.5075x over previous
import jax
import jax.numpy as jnp
from jax.experimental import pallas as pl

H = 512
W = 512
NPH = 32
NPW = 32
PH = H // NPH
PW = W // NPW
NP = NPH * NPW
C = 192

ROWF = PW * C
WCOLS = NPW * ROWF


def _perm_body(x_ref, o_ref):
    blk = x_ref[...].reshape(PH, NPW, ROWF)
    o_ref[...] = jnp.swapaxes(blk, 0, 1).reshape(NPW * PH, ROWF)


def kernel(x, mesh_pos, batch_idx):
    x2 = x.reshape(H, WCOLS)
    out = pl.pallas_call(
        _perm_body,
        grid=(NPH,),
        in_specs=[pl.BlockSpec((PH, WCOLS), lambda i: (i, 0))],
        out_specs=pl.BlockSpec((NPW * PH, ROWF), lambda i: (i, 0)),
        out_shape=jax.ShapeDtypeStruct((NP * PH, ROWF), jnp.float32),
    )(x2)
    return out.reshape(1, NP, PH, PW, C)
